# baseline (speedup 1.0000x reference)
import jax
import jax.numpy as jnp
from jax import lax
from jax.experimental import pallas as pl
from jax.experimental.pallas import tpu as pltpu
from jax.experimental.pallas import tpu_sc as plsc

N = 10000
E = 160000
H = 256
HH = 128
NG = 64
NB = 25
R = N // NB
NSUB = 16
NP = 10240
RPT = NP // NSUB
EPSUB = E // NSUB
ECH = 128
ECH_NCH = 80
BNEPS = 1e-5

def _sc_mesh():
    return plsc.VectorSubcoreMesh(core_axis_name="c", subcore_axis_name="s",
                                  num_cores=2, num_subcores=NSUB)


def _sc_deg_body(dst3_hbm, ones_hbm, zero_hbm, out_hbm, dstv, ones_v, ss, acc):
    c = lax.axis_index("c")
    s = lax.axis_index("s")
    pltpu.sync_copy(zero_hbm, acc.at[pl.ds(s * RPT, RPT)])
    pltpu.sync_copy(ones_hbm, ones_v)
    pltpu.sync_copy(dst3_hbm.at[s], dstv)
    plsc.subcore_barrier()
    nch = 40
    base = c * 40

    def chunk(j, carry):
        pltpu.async_copy(ones_v, acc.at[dstv.at[base + j]], ss, add=True)
        return carry

    lax.fori_loop(0, nch, chunk, 0)

    def drain(j, carry):
        pltpu.make_async_copy(ones_v, acc.at[dstv.at[0]], ss).wait()
        return carry

    lax.fori_loop(0, nch, drain, 0)
    plsc.subcore_barrier()
    pltpu.sync_copy(acc.at[pl.ds(s * RPT, RPT)],
                    out_hbm.at[pl.ds(c * NP + s * RPT, RPT)])


def _sc_deg(dst3, ones_rows, zero_rows):
    return pl.kernel(
        _sc_deg_body,
        jax.ShapeDtypeStruct((2 * NP, HH), jnp.float32),
        mesh=_sc_mesh(),
        scratch_types=[
            pltpu.VMEM((ECH_NCH, ECH), jnp.int32),
            pltpu.VMEM((ECH, HH), jnp.float32),
            pltpu.SemaphoreType.DMA,
            pltpu.VMEM_SHARED((NP, HH), jnp.float32),
        ],
    )(dst3, ones_rows, zero_rows)


def _sc_agg_body(ytab_hbm, srco_hbm, dst3_hbm, out_hbm,
                 dstv, idxq, rowsA, rowsB, gsA, gsB, ssA, ssB, acc):
    c = lax.axis_index("c")
    s = lax.axis_index("s")
    w = c * NSUB + s
    pltpu.sync_copy(ytab_hbm.at[pl.ds(c * NP + s * RPT, RPT)],
                    acc.at[pl.ds(s * RPT, RPT)])
    pltpu.sync_copy(dst3_hbm.at[s], dstv)
    plsc.subcore_barrier()

    def octet(o, carry):
        j0 = 8 * o
        pltpu.sync_copy(srco_hbm.at[pl.ds(w * ECH_NCH + j0, 8)], idxq)

        def sub(t, carry2):
            jA = j0 + 2 * t
            jB = jA + 1

            @pl.when((o > 0) | (t > 0))
            def _():
                pltpu.make_async_copy(rowsA, acc.at[dstv.at[jA]], ssA).wait()
            gA = pltpu.async_copy(ytab_hbm.at[idxq.at[2 * t]], rowsA, gsA)

            @pl.when((o > 0) | (t > 0))
            def _():
                pltpu.make_async_copy(rowsB, acc.at[dstv.at[jB]], ssB).wait()
            gB = pltpu.async_copy(ytab_hbm.at[idxq.at[2 * t + 1]], rowsB, gsB)
            gA.wait()
            pltpu.async_copy(rowsA, acc.at[dstv.at[jA]], ssA, add=True)
            gB.wait()
            pltpu.async_copy(rowsB, acc.at[dstv.at[jB]], ssB, add=True)
            return carry2

        lax.fori_loop(0, 4, sub, 0)
        return carry

    lax.fori_loop(0, ECH_NCH // 8, octet, 0)
    pltpu.make_async_copy(rowsA, acc.at[dstv.at[0]], ssA).wait()
    pltpu.make_async_copy(rowsB, acc.at[dstv.at[0]], ssB).wait()
    plsc.subcore_barrier()
    pltpu.sync_copy(acc.at[pl.ds(s * RPT, RPT)],
                    out_hbm.at[pl.ds(c * NP + s * RPT, RPT)])


def _sc_agg(ytab, srco, dst3):
    return pl.kernel(
        _sc_agg_body,
        jax.ShapeDtypeStruct((2 * NP, HH), jnp.float32),
        mesh=_sc_mesh(),
        scratch_types=[
            pltpu.VMEM((ECH_NCH, ECH), jnp.int32),
            pltpu.VMEM((8, ECH), jnp.int32),
            pltpu.VMEM((ECH, HH), jnp.float32),
            pltpu.VMEM((ECH, HH), jnp.float32),
            pltpu.SemaphoreType.DMA,
            pltpu.SemaphoreType.DMA,
            pltpu.SemaphoreType.DMA,
            pltpu.SemaphoreType.DMA,
            pltpu.VMEM_SHARED((NP, HH), jnp.float32),
        ],
    )(ytab, srco, dst3)


def _pad_edges(idx):
    per = idx.reshape(NSUB, EPSUB)
    pad = jnp.full((NSUB, ECH_NCH * ECH - EPSUB), NP - 1, jnp.int32)
    return jnp.concatenate([per, pad], axis=1).reshape(NSUB, ECH_NCH, ECH)


def _gelu(v):
    return 0.5 * v * (1.0 + lax.erf(v * 0.7071067811865476))


def _bn_apply(p, s_ref, q_ref, g_ref, be_ref):
    m = s_ref[...] * (1.0 / N)
    var = q_ref[...] * (1.0 / N) - m * m
    rstd = lax.rsqrt(var + BNEPS)
    return (p - m) * rstd * g_ref[...] + be_ref[...]


def _k1_body(x_ref, w_ref, b_ref, p_ref, s_ref, q_ref, accS, accQ):
    i = pl.program_id(0)
    p = jnp.dot(x_ref[...], w_ref[...], preferred_element_type=jnp.float32) + b_ref[...]
    p_ref[...] = p
    ps = jnp.sum(p, axis=0, keepdims=True)
    pq = jnp.sum(p * p, axis=0, keepdims=True)

    @pl.when(i == 0)
    def _():
        accS[...] = ps
        accQ[...] = pq

    @pl.when(i > 0)
    def _():
        accS[...] += ps
        accQ[...] += pq

    @pl.when(i == NB - 1)
    def _():
        s_ref[...] = accS[...]
        q_ref[...] = accQ[...]


def _k1(x, W_in, b_in):
    return pl.pallas_call(
        _k1_body,
        grid=(NB,),
        in_specs=[
            pl.BlockSpec((R, H), lambda i: (i, 0)),
            pl.BlockSpec((H, H), lambda i: (0, 0)),
            pl.BlockSpec((1, H), lambda i: (0, 0)),
        ],
        out_specs=[
            pl.BlockSpec((R, H), lambda i: (i, 0)),
            pl.BlockSpec((1, H), lambda i: (0, 0)),
            pl.BlockSpec((1, H), lambda i: (0, 0)),
        ],
        out_shape=[
            jax.ShapeDtypeStruct((N, H), jnp.float32),
            jax.ShapeDtypeStruct((1, H), jnp.float32),
            jax.ShapeDtypeStruct((1, H), jnp.float32),
        ],
        scratch_shapes=[
            pltpu.VMEM((1, H), jnp.float32),
            pltpu.VMEM((1, H), jnp.float32),
        ],
    )(x, W_in, b_in)


def _k3_body(p_ref, s_ref, q_ref, g_ref, be_ref, degA, degB, w_ref, batch_ref,
             h0_ref, y_ref, dinv_ref, xs_ref, cnt_ref, accXS, accCNT):
    i = pl.program_id(0)
    c = pl.program_id(1)
    h0 = _bn_apply(p_ref[...], s_ref, q_ref, g_ref, be_ref)
    d = degA[0, :, 0:1] + degB[0, :, 0:1] + 1.0
    dinv = lax.rsqrt(d)
    y = jnp.dot(h0, w_ref[...], preferred_element_type=jnp.float32) * dinv
    y_ref[0] = y

    @pl.when(c == 0)
    def _():
        h0_ref[...] = h0
        dinv_ref[...] = dinv
        onehot = (batch_ref[...] == lax.broadcasted_iota(jnp.int32, (1, NG), 1)
                  ).astype(jnp.float32)
        pxs = lax.dot_general(onehot, h0, (((0,), (0,)), ((), ())),
                              preferred_element_type=jnp.float32)
        pcnt = lax.dot_general(onehot, jnp.ones((R, 1), jnp.float32),
                               (((0,), (0,)), ((), ())),
                               preferred_element_type=jnp.float32)

        @pl.when(i == 0)
        def _():
            accXS[...] = pxs
            accCNT[...] = pcnt

        @pl.when(i > 0)
        def _():
            accXS[...] += pxs
            accCNT[...] += pcnt

    @pl.when((i == NB - 1) & (c == 1))
    def _():
        xs_ref[...] = accXS[...]
        cnt_ref[...] = accCNT[...]


def _k3(P, S, Q, g, be, degp, W1, batch2d):
    return pl.pallas_call(
        _k3_body,
        grid=(NB, 2),
        in_specs=[
            pl.BlockSpec((R, H), lambda i, c: (i, 0)),
            pl.BlockSpec((1, H), lambda i, c: (0, 0)),
            pl.BlockSpec((1, H), lambda i, c: (0, 0)),
            pl.BlockSpec((1, H), lambda i, c: (0, 0)),
            pl.BlockSpec((1, H), lambda i, c: (0, 0)),
            pl.BlockSpec((1, R, HH), lambda i, c: (0, i, 0)),
            pl.BlockSpec((1, R, HH), lambda i, c: (1, i, 0)),
            pl.BlockSpec((H, HH), lambda i, c: (0, c)),
            pl.BlockSpec((R, 1), lambda i, c: (i, 0)),
        ],
        out_specs=[
            pl.BlockSpec((R, H), lambda i, c: (i, 0)),
            pl.BlockSpec((1, R, HH), lambda i, c: (c, i, 0)),
            pl.BlockSpec((R, 1), lambda i, c: (i, 0)),
            pl.BlockSpec((NG, H), lambda i, c: (0, 0)),
            pl.BlockSpec((NG, 1), lambda i, c: (0, 0)),
        ],
        out_shape=[
            jax.ShapeDtypeStruct((N, H), jnp.float32),
            jax.ShapeDtypeStruct((2, NP, HH), jnp.float32),
            jax.ShapeDtypeStruct((N, 1), jnp.float32),
            jax.ShapeDtypeStruct((NG, H), jnp.float32),
            jax.ShapeDtypeStruct((NG, 1), jnp.float32),
        ],
        scratch_shapes=[
            pltpu.VMEM((NG, H), jnp.float32),
            pltpu.VMEM((NG, 1), jnp.float32),
        ],
    )(P, S, Q, g, be, degp, degp, W1, batch2d)


def _k5_body(aggA, aggB, dinv_ref, b_ref, c_ref, s_ref, q_ref, accS, accQ):
    i = pl.program_id(0)
    agg = jnp.concatenate([aggA[0], aggB[0]], axis=1)
    cp = agg * dinv_ref[...] + b_ref[...]
    c_ref[...] = cp
    ps = jnp.sum(cp, axis=0, keepdims=True)
    pq = jnp.sum(cp * cp, axis=0, keepdims=True)

    @pl.when(i == 0)
    def _():
        accS[...] = ps
        accQ[...] = pq

    @pl.when(i > 0)
    def _():
        accS[...] += ps
        accQ[...] += pq

    @pl.when(i == NB - 1)
    def _():
        s_ref[...] = accS[...]
        q_ref[...] = accQ[...]


def _k5(agg3, dinv, b):
    return pl.pallas_call(
        _k5_body,
        grid=(NB,),
        in_specs=[
            pl.BlockSpec((1, R, HH), lambda i: (0, i, 0)),
            pl.BlockSpec((1, R, HH), lambda i: (1, i, 0)),
            pl.BlockSpec((R, 1), lambda i: (i, 0)),
            pl.BlockSpec((1, H), lambda i: (0, 0)),
        ],
        out_specs=[
            pl.BlockSpec((R, H), lambda i: (i, 0)),
            pl.BlockSpec((1, H), lambda i: (0, 0)),
            pl.BlockSpec((1, H), lambda i: (0, 0)),
        ],
        out_shape=[
            jax.ShapeDtypeStruct((N, H), jnp.float32),
            jax.ShapeDtypeStruct((1, H), jnp.float32),
            jax.ShapeDtypeStruct((1, H), jnp.float32),
        ],
        scratch_shapes=[
            pltpu.VMEM((1, H), jnp.float32),
            pltpu.VMEM((1, H), jnp.float32),
        ],
    )(agg3, agg3, dinv, b)


def _k6_body(cp_ref, s_ref, q_ref, g_ref, be_ref, hprev_ref, w_ref, dinv_ref,
             h1_ref, y_ref):
    c = pl.program_id(1)
    hb = _gelu(_bn_apply(cp_ref[...], s_ref, q_ref, g_ref, be_ref))
    h1 = hb + hprev_ref[...]
    y = jnp.dot(h1, w_ref[...], preferred_element_type=jnp.float32) * dinv_ref[...]
    y_ref[0] = y

    @pl.when(c == 0)
    def _():
        h1_ref[...] = h1


def _k6(cp, S, Q, g, be, hprev, W, dinv):
    return pl.pallas_call(
        _k6_body,
        grid=(NB, 2),
        in_specs=[
            pl.BlockSpec((R, H), lambda i, c: (i, 0)),
            pl.BlockSpec((1, H), lambda i, c: (0, 0)),
            pl.BlockSpec((1, H), lambda i, c: (0, 0)),
            pl.BlockSpec((1, H), lambda i, c: (0, 0)),
            pl.BlockSpec((1, H), lambda i, c: (0, 0)),
            pl.BlockSpec((R, H), lambda i, c: (i, 0)),
            pl.BlockSpec((H, HH), lambda i, c: (0, c)),
            pl.BlockSpec((R, 1), lambda i, c: (i, 0)),
        ],
        out_specs=[
            pl.BlockSpec((R, H), lambda i, c: (i, 0)),
            pl.BlockSpec((1, R, HH), lambda i, c: (c, i, 0)),
        ],
        out_shape=[
            jax.ShapeDtypeStruct((N, H), jnp.float32),
            jax.ShapeDtypeStruct((2, NP, HH), jnp.float32),
        ],
    )(cp, S, Q, g, be, hprev, W, dinv)


def _k8b_body(cp_ref, s_ref, q_ref, g_ref, be_ref, hprev_ref, batch_ref,
              xg_ref, accXG):
    i = pl.program_id(0)
    hb = _gelu(_bn_apply(cp_ref[...], s_ref, q_ref, g_ref, be_ref))
    h2 = hb + hprev_ref[...]
    onehot = (batch_ref[...] == lax.broadcasted_iota(jnp.int32, (1, NG), 1)
              ).astype(jnp.float32)
    pxg = lax.dot_general(onehot, h2, (((0,), (0,)), ((), ())),
                          preferred_element_type=jnp.float32)

    @pl.when(i == 0)
    def _():
        accXG[...] = pxg

    @pl.when(i > 0)
    def _():
        accXG[...] += pxg

    @pl.when(i == NB - 1)
    def _():
        xg_ref[...] = accXG[...]


def _k8b(cp, S, Q, g, be, hprev, batch2d):
    return pl.pallas_call(
        _k8b_body,
        grid=(NB,),
        in_specs=[
            pl.BlockSpec((R, H), lambda i: (i, 0)),
            pl.BlockSpec((1, H), lambda i: (0, 0)),
            pl.BlockSpec((1, H), lambda i: (0, 0)),
            pl.BlockSpec((1, H), lambda i: (0, 0)),
            pl.BlockSpec((1, H), lambda i: (0, 0)),
            pl.BlockSpec((R, H), lambda i: (i, 0)),
            pl.BlockSpec((R, 1), lambda i: (i, 0)),
        ],
        out_specs=[pl.BlockSpec((NG, H), lambda i: (0, 0))],
        out_shape=[jax.ShapeDtypeStruct((NG, H), jnp.float32)],
        scratch_shapes=[pltpu.VMEM((NG, H), jnp.float32)],
    )(cp, S, Q, g, be, hprev, batch2d)


def _k9_body(xg_ref, xs_ref, cnt_ref, wf_ref, bf_ref, wc1_ref, bc1_ref,
             wc2_ref, bc2_ref, o_ref):
    inv = 1.0 / jnp.maximum(cnt_ref[...], 1.0)
    xg = xg_ref[...] * inv
    xs = xs_ref[...] * inv
    f = jnp.dot(xg, wf_ref[0:H], preferred_element_type=jnp.float32) + \
        jnp.dot(xs, wf_ref[H:2 * H], preferred_element_type=jnp.float32) + \
        bf_ref[...]
    f = _gelu(f)
    l1 = _gelu(jnp.dot(f, wc1_ref[...], preferred_element_type=jnp.float32)
               + bc1_ref[...])
    logits = jnp.dot(l1, wc2_ref[...], preferred_element_type=jnp.float32) \
        + bc2_ref[...]
    mx = jnp.max(logits, axis=1, keepdims=True)
    lse = jnp.log(jnp.sum(jnp.exp(logits - mx), axis=1, keepdims=True)) + mx
    o_ref[...] = logits - lse


def _k9(xg, xs, cnt, Wf, bf, Wc1, bc1, Wc2, bc2):
    return pl.pallas_call(
        _k9_body,
        out_shape=jax.ShapeDtypeStruct((NG, 20), jnp.float32),
    )(xg, xs, cnt, Wf, bf, Wc1, bc1, Wc2, bc2)


def kernel(x, edge_index, batch, W_in, b_in, g_in, be_in, W1, b1, g1, be1,
           W2, b2, g2, be2, Wf, bf, Wc1, bc1, Wc2, bc2):
    f32 = jnp.float32
    src = edge_index[0]
    dst = edge_index[1]
    batch2d = batch.reshape(N, 1)
    src3 = _pad_edges(src)
    dst3 = _pad_edges(dst)
    srco = jnp.concatenate([src3[None], src3[None] + NP],
                           axis=0).reshape(2 * NSUB * ECH_NCH, ECH)
    ones_rows = jnp.ones((ECH, HH), f32)
    zero_rows = jnp.zeros((RPT, HH), f32)

    P, S0, Q0 = _k1(x, W_in, b_in.reshape(1, H))

    degp = _sc_deg(dst3, ones_rows, zero_rows).reshape(2, NP, HH)

    h0, y1tab, dinv, xs_sum, cnt = _k3(
        P, S0, Q0, g_in.reshape(1, H), be_in.reshape(1, H), degp, W1, batch2d)

    agg1 = _sc_agg(y1tab.reshape(2 * NP, HH), srco, dst3).reshape(2, NP, HH)

    c1p, S1, Q1 = _k5(agg1, dinv, b1.reshape(1, H))

    h1, y2tab = _k6(c1p, S1, Q1, g1.reshape(1, H), be1.reshape(1, H), h0, W2,
                    dinv)

    agg2 = _sc_agg(y2tab.reshape(2 * NP, HH), srco, dst3).reshape(2, NP, HH)

    c2p, S2, Q2 = _k5(agg2, dinv, b2.reshape(1, H))

    xg_sum = _k8b(c2p, S2, Q2, g2.reshape(1, H), be2.reshape(1, H), h1,
                  batch2d)[0]

    return _k9(xg_sum, xs_sum, cnt, Wf, bf.reshape(1, H), Wc1,
               bc1.reshape(1, H // 2), Wc2, bc2.reshape(1, 20))


# speedup vs baseline: 1.0402x; 1.0402x over previous
import jax
import jax.numpy as jnp
from jax import lax
from jax.experimental import pallas as pl
from jax.experimental.pallas import tpu as pltpu
from jax.experimental.pallas import tpu_sc as plsc

N = 10000
E = 160000
H = 256
HH = 128
NG = 64
NB = 25
R = N // NB
NSUB = 16
NP = 10240
RPT = NP // NSUB
EPSUB = E // NSUB
ECH = 128
ECH_NCH = 80
BNEPS = 1e-5

def _sc_mesh():
    return plsc.VectorSubcoreMesh(core_axis_name="c", subcore_axis_name="s",
                                  num_cores=2, num_subcores=NSUB)


def _sc_deg_body(dst3_hbm, ones_hbm, zero_hbm, out_hbm, dstv, ones_v, ss, acc):
    c = lax.axis_index("c")
    s = lax.axis_index("s")
    pltpu.sync_copy(zero_hbm, acc.at[pl.ds(s * RPT, RPT)])
    pltpu.sync_copy(ones_hbm, ones_v)
    pltpu.sync_copy(dst3_hbm.at[s], dstv)
    plsc.subcore_barrier()
    nch = 40
    base = c * 40

    def chunk(j, carry):
        pltpu.async_copy(ones_v, acc.at[dstv.at[base + j]], ss, add=True)
        return carry

    lax.fori_loop(0, nch, chunk, 0)

    def drain(j, carry):
        pltpu.make_async_copy(ones_v, acc.at[dstv.at[0]], ss).wait()
        return carry

    lax.fori_loop(0, nch, drain, 0)
    plsc.subcore_barrier()
    pltpu.sync_copy(acc.at[pl.ds(s * RPT, RPT)],
                    out_hbm.at[pl.ds(c * NP + s * RPT, RPT)])


def _sc_deg(dst3, ones_rows, zero_rows):
    return pl.kernel(
        _sc_deg_body,
        jax.ShapeDtypeStruct((2 * NP, HH), jnp.float32),
        mesh=_sc_mesh(),
        scratch_types=[
            pltpu.VMEM((ECH_NCH, ECH), jnp.int32),
            pltpu.VMEM((ECH, HH), jnp.float32),
            pltpu.SemaphoreType.DMA,
            pltpu.VMEM_SHARED((NP, HH), jnp.float32),
        ],
    )(dst3, ones_rows, zero_rows)


def _sc_agg_body(ytab_hbm, srco_hbm, dst3_hbm, out_hbm,
                 dstv, srcA, srcB, rowsA, rowsB, gsA, gsB, ssA, ssB, acc):
    c = lax.axis_index("c")
    s = lax.axis_index("s")
    w = c * NSUB + s
    pltpu.sync_copy(ytab_hbm.at[pl.ds(c * NP + s * RPT, RPT)],
                    acc.at[pl.ds(s * RPT, RPT)])
    pltpu.sync_copy(dst3_hbm.at[s], dstv)
    plsc.subcore_barrier()

    def pair(g, carry):
        jA = 2 * g
        jB = jA + 1
        pltpu.sync_copy(srco_hbm.at[w * ECH_NCH + jA], srcA)

        @pl.when(g > 0)
        def _():
            pltpu.make_async_copy(rowsA, acc.at[dstv.at[jA]], ssA).wait()
        gA = pltpu.async_copy(ytab_hbm.at[srcA], rowsA, gsA)
        pltpu.sync_copy(srco_hbm.at[w * ECH_NCH + jB], srcB)

        @pl.when(g > 0)
        def _():
            pltpu.make_async_copy(rowsB, acc.at[dstv.at[jB]], ssB).wait()
        gB = pltpu.async_copy(ytab_hbm.at[srcB], rowsB, gsB)
        gA.wait()
        pltpu.async_copy(rowsA, acc.at[dstv.at[jA]], ssA, add=True)
        gB.wait()
        pltpu.async_copy(rowsB, acc.at[dstv.at[jB]], ssB, add=True)
        return carry

    lax.fori_loop(0, ECH_NCH // 2, pair, 0)
    pltpu.make_async_copy(rowsA, acc.at[dstv.at[0]], ssA).wait()
    pltpu.make_async_copy(rowsB, acc.at[dstv.at[0]], ssB).wait()
    plsc.subcore_barrier()
    pltpu.sync_copy(acc.at[pl.ds(s * RPT, RPT)],
                    out_hbm.at[pl.ds(c * NP + s * RPT, RPT)])


def _sc_agg(ytab, srco, dst3):
    return pl.kernel(
        _sc_agg_body,
        jax.ShapeDtypeStruct((2 * NP, HH), jnp.float32),
        mesh=_sc_mesh(),
        scratch_types=[
            pltpu.VMEM((ECH_NCH, ECH), jnp.int32),
            pltpu.VMEM((ECH,), jnp.int32),
            pltpu.VMEM((ECH,), jnp.int32),
            pltpu.VMEM((ECH, HH), jnp.float32),
            pltpu.VMEM((ECH, HH), jnp.float32),
            pltpu.SemaphoreType.DMA,
            pltpu.SemaphoreType.DMA,
            pltpu.SemaphoreType.DMA,
            pltpu.SemaphoreType.DMA,
            pltpu.VMEM_SHARED((NP, HH), jnp.float32),
        ],
    )(ytab, srco, dst3)


def _pad_edges(idx):
    per = idx.reshape(NSUB, EPSUB)
    pad = jnp.full((NSUB, ECH_NCH * ECH - EPSUB), NP - 1, jnp.int32)
    return jnp.concatenate([per, pad], axis=1).reshape(NSUB, ECH_NCH, ECH)


def _gelu(v):
    return 0.5 * v * (1.0 + lax.erf(v * 0.7071067811865476))


def _bn_apply(p, s_ref, q_ref, g_ref, be_ref):
    m = s_ref[...] * (1.0 / N)
    var = q_ref[...] * (1.0 / N) - m * m
    rstd = lax.rsqrt(var + BNEPS)
    return (p - m) * rstd * g_ref[...] + be_ref[...]


def _k1_body(x_ref, w_ref, b_ref, p_ref, s_ref, q_ref, accS, accQ):
    i = pl.program_id(0)
    p = jnp.dot(x_ref[...], w_ref[...], preferred_element_type=jnp.float32) + b_ref[...]
    p_ref[...] = p
    ps = jnp.sum(p, axis=0, keepdims=True)
    pq = jnp.sum(p * p, axis=0, keepdims=True)

    @pl.when(i == 0)
    def _():
        accS[...] = ps
        accQ[...] = pq

    @pl.when(i > 0)
    def _():
        accS[...] += ps
        accQ[...] += pq

    @pl.when(i == NB - 1)
    def _():
        s_ref[...] = accS[...]
        q_ref[...] = accQ[...]


def _k1(x, W_in, b_in):
    return pl.pallas_call(
        _k1_body,
        grid=(NB,),
        in_specs=[
            pl.BlockSpec((R, H), lambda i: (i, 0)),
            pl.BlockSpec((H, H), lambda i: (0, 0)),
            pl.BlockSpec((1, H), lambda i: (0, 0)),
        ],
        out_specs=[
            pl.BlockSpec((R, H), lambda i: (i, 0)),
            pl.BlockSpec((1, H), lambda i: (0, 0)),
            pl.BlockSpec((1, H), lambda i: (0, 0)),
        ],
        out_shape=[
            jax.ShapeDtypeStruct((N, H), jnp.float32),
            jax.ShapeDtypeStruct((1, H), jnp.float32),
            jax.ShapeDtypeStruct((1, H), jnp.float32),
        ],
        scratch_shapes=[
            pltpu.VMEM((1, H), jnp.float32),
            pltpu.VMEM((1, H), jnp.float32),
        ],
    )(x, W_in, b_in)


def _k3_body(p_ref, s_ref, q_ref, g_ref, be_ref, degA, degB, w_ref, batch_ref,
             h0_ref, y_ref, dinv_ref, xs_ref, cnt_ref, accXS, accCNT):
    i = pl.program_id(0)
    c = pl.program_id(1)
    h0 = _bn_apply(p_ref[...], s_ref, q_ref, g_ref, be_ref)
    d = degA[0, :, 0:1] + degB[0, :, 0:1] + 1.0
    dinv = lax.rsqrt(d)
    y = jnp.dot(h0, w_ref[...], preferred_element_type=jnp.float32) * dinv
    y_ref[0] = y

    @pl.when(c == 0)
    def _():
        h0_ref[...] = h0
        dinv_ref[...] = dinv
        onehot = (batch_ref[...] == lax.broadcasted_iota(jnp.int32, (1, NG), 1)
                  ).astype(jnp.float32)
        pxs = lax.dot_general(onehot, h0, (((0,), (0,)), ((), ())),
                              preferred_element_type=jnp.float32)
        pcnt = lax.dot_general(onehot, jnp.ones((R, 1), jnp.float32),
                               (((0,), (0,)), ((), ())),
                               preferred_element_type=jnp.float32)

        @pl.when(i == 0)
        def _():
            accXS[...] = pxs
            accCNT[...] = pcnt

        @pl.when(i > 0)
        def _():
            accXS[...] += pxs
            accCNT[...] += pcnt

    @pl.when((i == NB - 1) & (c == 1))
    def _():
        xs_ref[...] = accXS[...]
        cnt_ref[...] = accCNT[...]


def _k3(P, S, Q, g, be, degp, W1, batch2d):
    return pl.pallas_call(
        _k3_body,
        grid=(NB, 2),
        in_specs=[
            pl.BlockSpec((R, H), lambda i, c: (i, 0)),
            pl.BlockSpec((1, H), lambda i, c: (0, 0)),
            pl.BlockSpec((1, H), lambda i, c: (0, 0)),
            pl.BlockSpec((1, H), lambda i, c: (0, 0)),
            pl.BlockSpec((1, H), lambda i, c: (0, 0)),
            pl.BlockSpec((1, R, HH), lambda i, c: (0, i, 0)),
            pl.BlockSpec((1, R, HH), lambda i, c: (1, i, 0)),
            pl.BlockSpec((H, HH), lambda i, c: (0, c)),
            pl.BlockSpec((R, 1), lambda i, c: (i, 0)),
        ],
        out_specs=[
            pl.BlockSpec((R, H), lambda i, c: (i, 0)),
            pl.BlockSpec((1, R, HH), lambda i, c: (c, i, 0)),
            pl.BlockSpec((R, 1), lambda i, c: (i, 0)),
            pl.BlockSpec((NG, H), lambda i, c: (0, 0)),
            pl.BlockSpec((NG, 1), lambda i, c: (0, 0)),
        ],
        out_shape=[
            jax.ShapeDtypeStruct((N, H), jnp.float32),
            jax.ShapeDtypeStruct((2, NP, HH), jnp.float32),
            jax.ShapeDtypeStruct((N, 1), jnp.float32),
            jax.ShapeDtypeStruct((NG, H), jnp.float32),
            jax.ShapeDtypeStruct((NG, 1), jnp.float32),
        ],
        scratch_shapes=[
            pltpu.VMEM((NG, H), jnp.float32),
            pltpu.VMEM((NG, 1), jnp.float32),
        ],
    )(P, S, Q, g, be, degp, degp, W1, batch2d)


def _k5_body(aggA, aggB, dinv_ref, b_ref, c_ref, s_ref, q_ref, accS, accQ):
    i = pl.program_id(0)
    agg = jnp.concatenate([aggA[0], aggB[0]], axis=1)
    cp = agg * dinv_ref[...] + b_ref[...]
    c_ref[...] = cp
    ps = jnp.sum(cp, axis=0, keepdims=True)
    pq = jnp.sum(cp * cp, axis=0, keepdims=True)

    @pl.when(i == 0)
    def _():
        accS[...] = ps
        accQ[...] = pq

    @pl.when(i > 0)
    def _():
        accS[...] += ps
        accQ[...] += pq

    @pl.when(i == NB - 1)
    def _():
        s_ref[...] = accS[...]
        q_ref[...] = accQ[...]


def _k5(agg3, dinv, b):
    return pl.pallas_call(
        _k5_body,
        grid=(NB,),
        in_specs=[
            pl.BlockSpec((1, R, HH), lambda i: (0, i, 0)),
            pl.BlockSpec((1, R, HH), lambda i: (1, i, 0)),
            pl.BlockSpec((R, 1), lambda i: (i, 0)),
            pl.BlockSpec((1, H), lambda i: (0, 0)),
        ],
        out_specs=[
            pl.BlockSpec((R, H), lambda i: (i, 0)),
            pl.BlockSpec((1, H), lambda i: (0, 0)),
            pl.BlockSpec((1, H), lambda i: (0, 0)),
        ],
        out_shape=[
            jax.ShapeDtypeStruct((N, H), jnp.float32),
            jax.ShapeDtypeStruct((1, H), jnp.float32),
            jax.ShapeDtypeStruct((1, H), jnp.float32),
        ],
        scratch_shapes=[
            pltpu.VMEM((1, H), jnp.float32),
            pltpu.VMEM((1, H), jnp.float32),
        ],
    )(agg3, agg3, dinv, b)


def _k6_body(cp_ref, s_ref, q_ref, g_ref, be_ref, hprev_ref, w_ref, dinv_ref,
             h1_ref, y_ref):
    c = pl.program_id(1)
    hb = _gelu(_bn_apply(cp_ref[...], s_ref, q_ref, g_ref, be_ref))
    h1 = hb + hprev_ref[...]
    y = jnp.dot(h1, w_ref[...], preferred_element_type=jnp.float32) * dinv_ref[...]
    y_ref[0] = y

    @pl.when(c == 0)
    def _():
        h1_ref[...] = h1


def _k6(cp, S, Q, g, be, hprev, W, dinv):
    return pl.pallas_call(
        _k6_body,
        grid=(NB, 2),
        in_specs=[
            pl.BlockSpec((R, H), lambda i, c: (i, 0)),
            pl.BlockSpec((1, H), lambda i, c: (0, 0)),
            pl.BlockSpec((1, H), lambda i, c: (0, 0)),
            pl.BlockSpec((1, H), lambda i, c: (0, 0)),
            pl.BlockSpec((1, H), lambda i, c: (0, 0)),
            pl.BlockSpec((R, H), lambda i, c: (i, 0)),
            pl.BlockSpec((H, HH), lambda i, c: (0, c)),
            pl.BlockSpec((R, 1), lambda i, c: (i, 0)),
        ],
        out_specs=[
            pl.BlockSpec((R, H), lambda i, c: (i, 0)),
            pl.BlockSpec((1, R, HH), lambda i, c: (c, i, 0)),
        ],
        out_shape=[
            jax.ShapeDtypeStruct((N, H), jnp.float32),
            jax.ShapeDtypeStruct((2, NP, HH), jnp.float32),
        ],
    )(cp, S, Q, g, be, hprev, W, dinv)


def _k8b_body(cp_ref, s_ref, q_ref, g_ref, be_ref, hprev_ref, batch_ref,
              xg_ref, accXG):
    i = pl.program_id(0)
    hb = _gelu(_bn_apply(cp_ref[...], s_ref, q_ref, g_ref, be_ref))
    h2 = hb + hprev_ref[...]
    onehot = (batch_ref[...] == lax.broadcasted_iota(jnp.int32, (1, NG), 1)
              ).astype(jnp.float32)
    pxg = lax.dot_general(onehot, h2, (((0,), (0,)), ((), ())),
                          preferred_element_type=jnp.float32)

    @pl.when(i == 0)
    def _():
        accXG[...] = pxg

    @pl.when(i > 0)
    def _():
        accXG[...] += pxg

    @pl.when(i == NB - 1)
    def _():
        xg_ref[...] = accXG[...]


def _k8b(cp, S, Q, g, be, hprev, batch2d):
    return pl.pallas_call(
        _k8b_body,
        grid=(NB,),
        in_specs=[
            pl.BlockSpec((R, H), lambda i: (i, 0)),
            pl.BlockSpec((1, H), lambda i: (0, 0)),
            pl.BlockSpec((1, H), lambda i: (0, 0)),
            pl.BlockSpec((1, H), lambda i: (0, 0)),
            pl.BlockSpec((1, H), lambda i: (0, 0)),
            pl.BlockSpec((R, H), lambda i: (i, 0)),
            pl.BlockSpec((R, 1), lambda i: (i, 0)),
        ],
        out_specs=[pl.BlockSpec((NG, H), lambda i: (0, 0))],
        out_shape=[jax.ShapeDtypeStruct((NG, H), jnp.float32)],
        scratch_shapes=[pltpu.VMEM((NG, H), jnp.float32)],
    )(cp, S, Q, g, be, hprev, batch2d)


def _k9_body(xg_ref, xs_ref, cnt_ref, wf_ref, bf_ref, wc1_ref, bc1_ref,
             wc2_ref, bc2_ref, o_ref):
    inv = 1.0 / jnp.maximum(cnt_ref[...], 1.0)
    xg = xg_ref[...] * inv
    xs = xs_ref[...] * inv
    f = jnp.dot(xg, wf_ref[0:H], preferred_element_type=jnp.float32) + \
        jnp.dot(xs, wf_ref[H:2 * H], preferred_element_type=jnp.float32) + \
        bf_ref[...]
    f = _gelu(f)
    l1 = _gelu(jnp.dot(f, wc1_ref[...], preferred_element_type=jnp.float32)
               + bc1_ref[...])
    logits = jnp.dot(l1, wc2_ref[...], preferred_element_type=jnp.float32) \
        + bc2_ref[...]
    mx = jnp.max(logits, axis=1, keepdims=True)
    lse = jnp.log(jnp.sum(jnp.exp(logits - mx), axis=1, keepdims=True)) + mx
    o_ref[...] = logits - lse


def _k9(xg, xs, cnt, Wf, bf, Wc1, bc1, Wc2, bc2):
    return pl.pallas_call(
        _k9_body,
        out_shape=jax.ShapeDtypeStruct((NG, 20), jnp.float32),
    )(xg, xs, cnt, Wf, bf, Wc1, bc1, Wc2, bc2)


def kernel(x, edge_index, batch, W_in, b_in, g_in, be_in, W1, b1, g1, be1,
           W2, b2, g2, be2, Wf, bf, Wc1, bc1, Wc2, bc2):
    f32 = jnp.float32
    src = edge_index[0]
    dst = edge_index[1]
    batch2d = batch.reshape(N, 1)
    src3 = _pad_edges(src)
    dst3 = _pad_edges(dst)
    srco = jnp.concatenate([src3[None], src3[None] + NP],
                           axis=0).reshape(2 * NSUB * ECH_NCH, ECH)
    ones_rows = jnp.ones((ECH, HH), f32)
    zero_rows = jnp.zeros((RPT, HH), f32)

    P, S0, Q0 = _k1(x, W_in, b_in.reshape(1, H))

    degp = _sc_deg(dst3, ones_rows, zero_rows).reshape(2, NP, HH)

    h0, y1tab, dinv, xs_sum, cnt = _k3(
        P, S0, Q0, g_in.reshape(1, H), be_in.reshape(1, H), degp, W1, batch2d)

    agg1 = _sc_agg(y1tab.reshape(2 * NP, HH), srco, dst3).reshape(2, NP, HH)

    c1p, S1, Q1 = _k5(agg1, dinv, b1.reshape(1, H))

    h1, y2tab = _k6(c1p, S1, Q1, g1.reshape(1, H), be1.reshape(1, H), h0, W2,
                    dinv)

    agg2 = _sc_agg(y2tab.reshape(2 * NP, HH), srco, dst3).reshape(2, NP, HH)

    c2p, S2, Q2 = _k5(agg2, dinv, b2.reshape(1, H))

    xg_sum = _k8b(c2p, S2, Q2, g2.reshape(1, H), be2.reshape(1, H), h1,
                  batch2d)[0]

    return _k9(xg_sum, xs_sum, cnt, Wf, bf.reshape(1, H), Wc1,
               bc1.reshape(1, H // 2), Wc2, bc2.reshape(1, 20))


# speedup vs baseline: 1.6152x; 1.5527x over previous
import jax
import jax.numpy as jnp
from jax import lax
from jax.experimental import pallas as pl
from jax.experimental.pallas import tpu as pltpu
from jax.experimental.pallas import tpu_sc as plsc

N = 10000
E = 160000
H = 256
HH = 128
NG = 64
NB = 25
R = N // NB
NSUB = 16
NP = 10240
RPT = NP // NSUB
EPSUB = E // NSUB
ECH = 128
ECH_NCH = 80
BNEPS = 1e-5

def _sc_mesh():
    return plsc.VectorSubcoreMesh(core_axis_name="c", subcore_axis_name="s",
                                  num_cores=2, num_subcores=NSUB)


def _sc_deg_body(dst3_hbm, ones_hbm, zero_hbm, out_hbm, dstv, ones_v, ss, acc):
    c = lax.axis_index("c")
    s = lax.axis_index("s")
    pltpu.sync_copy(zero_hbm, acc.at[pl.ds(s * RPT, RPT)])
    pltpu.sync_copy(ones_hbm, ones_v)
    pltpu.sync_copy(dst3_hbm.at[s], dstv)
    plsc.subcore_barrier()
    nch = 40
    base = c * 40

    def chunk(j, carry):
        pltpu.async_copy(ones_v, acc.at[dstv.at[base + j]], ss, add=True)
        return carry

    lax.fori_loop(0, nch, chunk, 0)

    def drain(j, carry):
        pltpu.make_async_copy(ones_v, acc.at[dstv.at[0]], ss).wait()
        return carry

    lax.fori_loop(0, nch, drain, 0)
    plsc.subcore_barrier()
    pltpu.sync_copy(acc.at[pl.ds(s * RPT, RPT)],
                    out_hbm.at[pl.ds(c * NP + s * RPT, RPT)])


def _sc_deg(dst3, ones_rows, zero_rows):
    return pl.kernel(
        _sc_deg_body,
        jax.ShapeDtypeStruct((2 * NP, HH), jnp.float32),
        mesh=_sc_mesh(),
        scratch_types=[
            pltpu.VMEM((ECH_NCH, ECH), jnp.int32),
            pltpu.VMEM((ECH, HH), jnp.float32),
            pltpu.SemaphoreType.DMA,
            pltpu.VMEM_SHARED((NP, HH), jnp.float32),
        ],
    )(dst3, ones_rows, zero_rows)


def _sc_agg_body(ytab_hbm, srco_hbm, dst3_hbm, out_hbm,
                 dstv, srcA, srcB, rowsA, rowsB, gsA, gsB, ssA, ssB, acc):
    c = lax.axis_index("c")
    s = lax.axis_index("s")
    w = c * NSUB + s
    pltpu.sync_copy(ytab_hbm.at[pl.ds(c * NP + s * RPT, RPT)],
                    acc.at[pl.ds(s * RPT, RPT)])
    pltpu.sync_copy(dst3_hbm.at[s], dstv)
    plsc.subcore_barrier()

    def pair(g, carry):
        jA = 2 * g
        jB = jA + 1
        pltpu.sync_copy(srco_hbm.at[w * ECH_NCH + jA], srcA)

        @pl.when(g > 0)
        def _():
            pltpu.make_async_copy(rowsA, acc.at[dstv.at[jA]], ssA).wait()
        gA = pltpu.async_copy(ytab_hbm.at[srcA], rowsA, gsA)
        pltpu.sync_copy(srco_hbm.at[w * ECH_NCH + jB], srcB)

        @pl.when(g > 0)
        def _():
            pltpu.make_async_copy(rowsB, acc.at[dstv.at[jB]], ssB).wait()
        gB = pltpu.async_copy(ytab_hbm.at[srcB], rowsB, gsB)
        gA.wait()
        pltpu.async_copy(rowsA, acc.at[dstv.at[jA]], ssA, add=True)
        gB.wait()
        pltpu.async_copy(rowsB, acc.at[dstv.at[jB]], ssB, add=True)
        return carry

    lax.fori_loop(0, ECH_NCH // 2, pair, 0)
    pltpu.make_async_copy(rowsA, acc.at[dstv.at[0]], ssA).wait()
    pltpu.make_async_copy(rowsB, acc.at[dstv.at[0]], ssB).wait()
    plsc.subcore_barrier()
    pltpu.sync_copy(acc.at[pl.ds(s * RPT, RPT)],
                    out_hbm.at[pl.ds(c * NP + s * RPT, RPT)])


def _sc_agg(ytab, srco, dst3):
    return pl.kernel(
        _sc_agg_body,
        jax.ShapeDtypeStruct((2 * NP, HH), jnp.float32),
        mesh=_sc_mesh(),
        scratch_types=[
            pltpu.VMEM((ECH_NCH, ECH), jnp.int32),
            pltpu.VMEM((ECH,), jnp.int32),
            pltpu.VMEM((ECH,), jnp.int32),
            pltpu.VMEM((ECH, HH), jnp.float32),
            pltpu.VMEM((ECH, HH), jnp.float32),
            pltpu.SemaphoreType.DMA,
            pltpu.SemaphoreType.DMA,
            pltpu.SemaphoreType.DMA,
            pltpu.SemaphoreType.DMA,
            pltpu.VMEM_SHARED((NP, HH), jnp.float32),
        ],
    )(ytab, srco, dst3)


def _pad_edges(idx):
    per = idx.reshape(NSUB, EPSUB)
    npad = ECH_NCH * ECH - EPSUB
    padv = N + (jnp.arange(npad, dtype=jnp.int32) % (NP - N))
    pad = jnp.broadcast_to(padv, (NSUB, npad))
    return jnp.concatenate([per, pad], axis=1).reshape(NSUB, ECH_NCH, ECH)


def _gelu(v):
    return 0.5 * v * (1.0 + lax.erf(v * 0.7071067811865476))


def _bn_apply(p, s_ref, q_ref, g_ref, be_ref):
    m = s_ref[...] * (1.0 / N)
    var = q_ref[...] * (1.0 / N) - m * m
    rstd = lax.rsqrt(var + BNEPS)
    return (p - m) * rstd * g_ref[...] + be_ref[...]


def _k1_body(x_ref, w_ref, b_ref, p_ref, s_ref, q_ref, accS, accQ):
    i = pl.program_id(0)
    p = jnp.dot(x_ref[...], w_ref[...], preferred_element_type=jnp.float32) + b_ref[...]
    p_ref[...] = p
    ps = jnp.sum(p, axis=0, keepdims=True)
    pq = jnp.sum(p * p, axis=0, keepdims=True)

    @pl.when(i == 0)
    def _():
        accS[...] = ps
        accQ[...] = pq

    @pl.when(i > 0)
    def _():
        accS[...] += ps
        accQ[...] += pq

    @pl.when(i == NB - 1)
    def _():
        s_ref[...] = accS[...]
        q_ref[...] = accQ[...]


def _k1(x, W_in, b_in):
    return pl.pallas_call(
        _k1_body,
        grid=(NB,),
        in_specs=[
            pl.BlockSpec((R, H), lambda i: (i, 0)),
            pl.BlockSpec((H, H), lambda i: (0, 0)),
            pl.BlockSpec((1, H), lambda i: (0, 0)),
        ],
        out_specs=[
            pl.BlockSpec((R, H), lambda i: (i, 0)),
            pl.BlockSpec((1, H), lambda i: (0, 0)),
            pl.BlockSpec((1, H), lambda i: (0, 0)),
        ],
        out_shape=[
            jax.ShapeDtypeStruct((N, H), jnp.float32),
            jax.ShapeDtypeStruct((1, H), jnp.float32),
            jax.ShapeDtypeStruct((1, H), jnp.float32),
        ],
        scratch_shapes=[
            pltpu.VMEM((1, H), jnp.float32),
            pltpu.VMEM((1, H), jnp.float32),
        ],
    )(x, W_in, b_in)


def _k3_body(p_ref, s_ref, q_ref, g_ref, be_ref, degA, degB, w_ref, batch_ref,
             h0_ref, y_ref, dinv_ref, xs_ref, cnt_ref, accXS, accCNT):
    i = pl.program_id(0)
    c = pl.program_id(1)
    h0 = _bn_apply(p_ref[...], s_ref, q_ref, g_ref, be_ref)
    d = degA[0, :, 0:1] + degB[0, :, 0:1] + 1.0
    dinv = lax.rsqrt(d)
    y = jnp.dot(h0, w_ref[...], preferred_element_type=jnp.float32) * dinv
    y_ref[0] = y

    @pl.when(c == 0)
    def _():
        h0_ref[...] = h0
        dinv_ref[...] = dinv
        onehot = (batch_ref[...] == lax.broadcasted_iota(jnp.int32, (1, NG), 1)
                  ).astype(jnp.float32)
        pxs = lax.dot_general(onehot, h0, (((0,), (0,)), ((), ())),
                              preferred_element_type=jnp.float32)
        pcnt = lax.dot_general(onehot, jnp.ones((R, 1), jnp.float32),
                               (((0,), (0,)), ((), ())),
                               preferred_element_type=jnp.float32)

        @pl.when(i == 0)
        def _():
            accXS[...] = pxs
            accCNT[...] = pcnt

        @pl.when(i > 0)
        def _():
            accXS[...] += pxs
            accCNT[...] += pcnt

    @pl.when((i == NB - 1) & (c == 1))
    def _():
        xs_ref[...] = accXS[...]
        cnt_ref[...] = accCNT[...]


def _k3(P, S, Q, g, be, degp, W1, batch2d):
    return pl.pallas_call(
        _k3_body,
        grid=(NB, 2),
        in_specs=[
            pl.BlockSpec((R, H), lambda i, c: (i, 0)),
            pl.BlockSpec((1, H), lambda i, c: (0, 0)),
            pl.BlockSpec((1, H), lambda i, c: (0, 0)),
            pl.BlockSpec((1, H), lambda i, c: (0, 0)),
            pl.BlockSpec((1, H), lambda i, c: (0, 0)),
            pl.BlockSpec((1, R, HH), lambda i, c: (0, i, 0)),
            pl.BlockSpec((1, R, HH), lambda i, c: (1, i, 0)),
            pl.BlockSpec((H, HH), lambda i, c: (0, c)),
            pl.BlockSpec((R, 1), lambda i, c: (i, 0)),
        ],
        out_specs=[
            pl.BlockSpec((R, H), lambda i, c: (i, 0)),
            pl.BlockSpec((1, R, HH), lambda i, c: (c, i, 0)),
            pl.BlockSpec((R, 1), lambda i, c: (i, 0)),
            pl.BlockSpec((NG, H), lambda i, c: (0, 0)),
            pl.BlockSpec((NG, 1), lambda i, c: (0, 0)),
        ],
        out_shape=[
            jax.ShapeDtypeStruct((N, H), jnp.float32),
            jax.ShapeDtypeStruct((2, NP, HH), jnp.float32),
            jax.ShapeDtypeStruct((N, 1), jnp.float32),
            jax.ShapeDtypeStruct((NG, H), jnp.float32),
            jax.ShapeDtypeStruct((NG, 1), jnp.float32),
        ],
        scratch_shapes=[
            pltpu.VMEM((NG, H), jnp.float32),
            pltpu.VMEM((NG, 1), jnp.float32),
        ],
    )(P, S, Q, g, be, degp, degp, W1, batch2d)


def _k5_body(aggA, aggB, dinv_ref, b_ref, c_ref, s_ref, q_ref, accS, accQ):
    i = pl.program_id(0)
    agg = jnp.concatenate([aggA[0], aggB[0]], axis=1)
    cp = agg * dinv_ref[...] + b_ref[...]
    c_ref[...] = cp
    ps = jnp.sum(cp, axis=0, keepdims=True)
    pq = jnp.sum(cp * cp, axis=0, keepdims=True)

    @pl.when(i == 0)
    def _():
        accS[...] = ps
        accQ[...] = pq

    @pl.when(i > 0)
    def _():
        accS[...] += ps
        accQ[...] += pq

    @pl.when(i == NB - 1)
    def _():
        s_ref[...] = accS[...]
        q_ref[...] = accQ[...]


def _k5(agg3, dinv, b):
    return pl.pallas_call(
        _k5_body,
        grid=(NB,),
        in_specs=[
            pl.BlockSpec((1, R, HH), lambda i: (0, i, 0)),
            pl.BlockSpec((1, R, HH), lambda i: (1, i, 0)),
            pl.BlockSpec((R, 1), lambda i: (i, 0)),
            pl.BlockSpec((1, H), lambda i: (0, 0)),
        ],
        out_specs=[
            pl.BlockSpec((R, H), lambda i: (i, 0)),
            pl.BlockSpec((1, H), lambda i: (0, 0)),
            pl.BlockSpec((1, H), lambda i: (0, 0)),
        ],
        out_shape=[
            jax.ShapeDtypeStruct((N, H), jnp.float32),
            jax.ShapeDtypeStruct((1, H), jnp.float32),
            jax.ShapeDtypeStruct((1, H), jnp.float32),
        ],
        scratch_shapes=[
            pltpu.VMEM((1, H), jnp.float32),
            pltpu.VMEM((1, H), jnp.float32),
        ],
    )(agg3, agg3, dinv, b)


def _k6_body(cp_ref, s_ref, q_ref, g_ref, be_ref, hprev_ref, w_ref, dinv_ref,
             h1_ref, y_ref):
    c = pl.program_id(1)
    hb = _gelu(_bn_apply(cp_ref[...], s_ref, q_ref, g_ref, be_ref))
    h1 = hb + hprev_ref[...]
    y = jnp.dot(h1, w_ref[...], preferred_element_type=jnp.float32) * dinv_ref[...]
    y_ref[0] = y

    @pl.when(c == 0)
    def _():
        h1_ref[...] = h1


def _k6(cp, S, Q, g, be, hprev, W, dinv):
    return pl.pallas_call(
        _k6_body,
        grid=(NB, 2),
        in_specs=[
            pl.BlockSpec((R, H), lambda i, c: (i, 0)),
            pl.BlockSpec((1, H), lambda i, c: (0, 0)),
            pl.BlockSpec((1, H), lambda i, c: (0, 0)),
            pl.BlockSpec((1, H), lambda i, c: (0, 0)),
            pl.BlockSpec((1, H), lambda i, c: (0, 0)),
            pl.BlockSpec((R, H), lambda i, c: (i, 0)),
            pl.BlockSpec((H, HH), lambda i, c: (0, c)),
            pl.BlockSpec((R, 1), lambda i, c: (i, 0)),
        ],
        out_specs=[
            pl.BlockSpec((R, H), lambda i, c: (i, 0)),
            pl.BlockSpec((1, R, HH), lambda i, c: (c, i, 0)),
        ],
        out_shape=[
            jax.ShapeDtypeStruct((N, H), jnp.float32),
            jax.ShapeDtypeStruct((2, NP, HH), jnp.float32),
        ],
    )(cp, S, Q, g, be, hprev, W, dinv)


def _k8b_body(cp_ref, s_ref, q_ref, g_ref, be_ref, hprev_ref, batch_ref,
              xg_ref, accXG):
    i = pl.program_id(0)
    hb = _gelu(_bn_apply(cp_ref[...], s_ref, q_ref, g_ref, be_ref))
    h2 = hb + hprev_ref[...]
    onehot = (batch_ref[...] == lax.broadcasted_iota(jnp.int32, (1, NG), 1)
              ).astype(jnp.float32)
    pxg = lax.dot_general(onehot, h2, (((0,), (0,)), ((), ())),
                          preferred_element_type=jnp.float32)

    @pl.when(i == 0)
    def _():
        accXG[...] = pxg

    @pl.when(i > 0)
    def _():
        accXG[...] += pxg

    @pl.when(i == NB - 1)
    def _():
        xg_ref[...] = accXG[...]


def _k8b(cp, S, Q, g, be, hprev, batch2d):
    return pl.pallas_call(
        _k8b_body,
        grid=(NB,),
        in_specs=[
            pl.BlockSpec((R, H), lambda i: (i, 0)),
            pl.BlockSpec((1, H), lambda i: (0, 0)),
            pl.BlockSpec((1, H), lambda i: (0, 0)),
            pl.BlockSpec((1, H), lambda i: (0, 0)),
            pl.BlockSpec((1, H), lambda i: (0, 0)),
            pl.BlockSpec((R, H), lambda i: (i, 0)),
            pl.BlockSpec((R, 1), lambda i: (i, 0)),
        ],
        out_specs=[pl.BlockSpec((NG, H), lambda i: (0, 0))],
        out_shape=[jax.ShapeDtypeStruct((NG, H), jnp.float32)],
        scratch_shapes=[pltpu.VMEM((NG, H), jnp.float32)],
    )(cp, S, Q, g, be, hprev, batch2d)


def _k9_body(xg_ref, xs_ref, cnt_ref, wf_ref, bf_ref, wc1_ref, bc1_ref,
             wc2_ref, bc2_ref, o_ref):
    inv = 1.0 / jnp.maximum(cnt_ref[...], 1.0)
    xg = xg_ref[...] * inv
    xs = xs_ref[...] * inv
    f = jnp.dot(xg, wf_ref[0:H], preferred_element_type=jnp.float32) + \
        jnp.dot(xs, wf_ref[H:2 * H], preferred_element_type=jnp.float32) + \
        bf_ref[...]
    f = _gelu(f)
    l1 = _gelu(jnp.dot(f, wc1_ref[...], preferred_element_type=jnp.float32)
               + bc1_ref[...])
    logits = jnp.dot(l1, wc2_ref[...], preferred_element_type=jnp.float32) \
        + bc2_ref[...]
    mx = jnp.max(logits, axis=1, keepdims=True)
    lse = jnp.log(jnp.sum(jnp.exp(logits - mx), axis=1, keepdims=True)) + mx
    o_ref[...] = logits - lse


def _k9(xg, xs, cnt, Wf, bf, Wc1, bc1, Wc2, bc2):
    return pl.pallas_call(
        _k9_body,
        out_shape=jax.ShapeDtypeStruct((NG, 20), jnp.float32),
    )(xg, xs, cnt, Wf, bf, Wc1, bc1, Wc2, bc2)


def kernel(x, edge_index, batch, W_in, b_in, g_in, be_in, W1, b1, g1, be1,
           W2, b2, g2, be2, Wf, bf, Wc1, bc1, Wc2, bc2):
    f32 = jnp.float32
    src = edge_index[0]
    dst = edge_index[1]
    batch2d = batch.reshape(N, 1)
    src3 = _pad_edges(src)
    dst3 = _pad_edges(dst)
    srco = jnp.concatenate([src3[None], src3[None] + NP],
                           axis=0).reshape(2 * NSUB * ECH_NCH, ECH)
    ones_rows = jnp.ones((ECH, HH), f32)
    zero_rows = jnp.zeros((RPT, HH), f32)

    P, S0, Q0 = _k1(x, W_in, b_in.reshape(1, H))

    degp = _sc_deg(dst3, ones_rows, zero_rows).reshape(2, NP, HH)

    h0, y1tab, dinv, xs_sum, cnt = _k3(
        P, S0, Q0, g_in.reshape(1, H), be_in.reshape(1, H), degp, W1, batch2d)

    agg1 = _sc_agg(y1tab.reshape(2 * NP, HH), srco, dst3).reshape(2, NP, HH)

    c1p, S1, Q1 = _k5(agg1, dinv, b1.reshape(1, H))

    h1, y2tab = _k6(c1p, S1, Q1, g1.reshape(1, H), be1.reshape(1, H), h0, W2,
                    dinv)

    agg2 = _sc_agg(y2tab.reshape(2 * NP, HH), srco, dst3).reshape(2, NP, HH)

    c2p, S2, Q2 = _k5(agg2, dinv, b2.reshape(1, H))

    xg_sum = _k8b(c2p, S2, Q2, g2.reshape(1, H), be2.reshape(1, H), h1,
                  batch2d)[0]

    return _k9(xg_sum, xs_sum, cnt, Wf, bf.reshape(1, H), Wc1,
               bc1.reshape(1, H // 2), Wc2, bc2.reshape(1, 20))


# speedup vs baseline: 1.6246x; 1.0058x over previous
import jax
import jax.numpy as jnp
from jax import lax
from jax.experimental import pallas as pl
from jax.experimental.pallas import tpu as pltpu
from jax.experimental.pallas import tpu_sc as plsc

N = 10000
E = 160000
H = 256
HH = 128
NG = 64
NB = 25
R = N // NB
NSUB = 16
NP = 10240
RPT = NP // NSUB
EPSUB = E // NSUB
ECH = 128
ECH_NCH = 80
BNEPS = 1e-5

def _sc_mesh():
    return plsc.VectorSubcoreMesh(core_axis_name="c", subcore_axis_name="s",
                                  num_cores=2, num_subcores=NSUB)


def _sc_deg_body(dst3_hbm, ones_hbm, zero_hbm, out_hbm, dstv, ones_v, ss, acc):
    c = lax.axis_index("c")
    s = lax.axis_index("s")
    pltpu.sync_copy(zero_hbm, acc.at[pl.ds(s * RPT, RPT)])
    pltpu.sync_copy(ones_hbm, ones_v)
    pltpu.sync_copy(dst3_hbm.at[s], dstv)
    plsc.subcore_barrier()
    nch = 40
    base = c * 40

    def chunk(j, carry):
        pltpu.async_copy(ones_v, acc.at[dstv.at[base + j]], ss, add=True)
        return carry

    lax.fori_loop(0, nch, chunk, 0)

    def drain(j, carry):
        pltpu.make_async_copy(ones_v, acc.at[dstv.at[0]], ss).wait()
        return carry

    lax.fori_loop(0, nch, drain, 0)
    plsc.subcore_barrier()
    pltpu.sync_copy(acc.at[pl.ds(s * RPT, RPT)],
                    out_hbm.at[pl.ds(c * NP + s * RPT, RPT)])


def _sc_deg(dst3, ones_rows, zero_rows):
    return pl.kernel(
        _sc_deg_body,
        jax.ShapeDtypeStruct((2 * NP, HH), jnp.float32),
        mesh=_sc_mesh(),
        scratch_types=[
            pltpu.VMEM((ECH_NCH, ECH), jnp.int32),
            pltpu.VMEM((ECH, HH), jnp.float32),
            pltpu.SemaphoreType.DMA,
            pltpu.VMEM_SHARED((NP, HH), jnp.float32),
        ],
    )(dst3, ones_rows, zero_rows)


def _sc_agg_body(ytab_hbm, srco_hbm, dst3_hbm, out_hbm,
                 dstv, srcA, srcB, rowsA, rowsB, gsA, gsB, ssA, ssB, acc):
    c = lax.axis_index("c")
    s = lax.axis_index("s")
    w = c * NSUB + s
    pltpu.sync_copy(ytab_hbm.at[pl.ds(c * NP + s * RPT, RPT)],
                    acc.at[pl.ds(s * RPT, RPT)])
    pltpu.sync_copy(dst3_hbm.at[s], dstv)
    plsc.subcore_barrier()

    def pair(g, carry):
        jA = 2 * g
        jB = jA + 1
        pltpu.sync_copy(srco_hbm.at[w * ECH_NCH + jA], srcA)

        @pl.when(g > 0)
        def _():
            pltpu.make_async_copy(rowsA, acc.at[dstv.at[jA]], ssA).wait()
        gA = pltpu.async_copy(ytab_hbm.at[srcA], rowsA, gsA)
        pltpu.sync_copy(srco_hbm.at[w * ECH_NCH + jB], srcB)

        @pl.when(g > 0)
        def _():
            pltpu.make_async_copy(rowsB, acc.at[dstv.at[jB]], ssB).wait()
        gB = pltpu.async_copy(ytab_hbm.at[srcB], rowsB, gsB)
        gA.wait()
        pltpu.async_copy(rowsA, acc.at[dstv.at[jA]], ssA, add=True)
        gB.wait()
        pltpu.async_copy(rowsB, acc.at[dstv.at[jB]], ssB, add=True)
        return carry

    lax.fori_loop(0, ECH_NCH // 2, pair, 0)
    pltpu.make_async_copy(rowsA, acc.at[dstv.at[0]], ssA).wait()
    pltpu.make_async_copy(rowsB, acc.at[dstv.at[0]], ssB).wait()
    plsc.subcore_barrier()
    pltpu.sync_copy(acc.at[pl.ds(s * RPT, RPT)],
                    out_hbm.at[pl.ds(c * NP + s * RPT, RPT)])


def _sc_agg(ytab, srco, dst3):
    return pl.kernel(
        _sc_agg_body,
        jax.ShapeDtypeStruct((2 * NP, HH), jnp.float32),
        mesh=_sc_mesh(),
        scratch_types=[
            pltpu.VMEM((ECH_NCH, ECH), jnp.int32),
            pltpu.VMEM((ECH,), jnp.int32),
            pltpu.VMEM((ECH,), jnp.int32),
            pltpu.VMEM((ECH, HH), jnp.float32),
            pltpu.VMEM((ECH, HH), jnp.float32),
            pltpu.SemaphoreType.DMA,
            pltpu.SemaphoreType.DMA,
            pltpu.SemaphoreType.DMA,
            pltpu.SemaphoreType.DMA,
            pltpu.VMEM_SHARED((NP, HH), jnp.float32),
        ],
    )(ytab, srco, dst3)


def _pad_edges(idx):
    per = idx.reshape(NSUB, EPSUB)
    npad = ECH_NCH * ECH - EPSUB
    padv = N + (jnp.arange(npad, dtype=jnp.int32) % (NP - N))
    pad = jnp.broadcast_to(padv, (NSUB, npad))
    return jnp.concatenate([per, pad], axis=1).reshape(NSUB, ECH_NCH, ECH)


def _gelu(v):
    return 0.5 * v * (1.0 + lax.erf(v * 0.7071067811865476))


def _bn_apply(p, s_ref, q_ref, g_ref, be_ref):
    m = s_ref[...] * (1.0 / N)
    var = q_ref[...] * (1.0 / N) - m * m
    rstd = lax.rsqrt(var + BNEPS)
    return (p - m) * rstd * g_ref[...] + be_ref[...]


def _k1_body(x_ref, w_ref, b_ref, p_ref, s_ref, q_ref, accS, accQ):
    i = pl.program_id(0)
    p = jnp.dot(x_ref[...], w_ref[...], preferred_element_type=jnp.float32) + b_ref[...]
    p_ref[...] = p
    ps = jnp.sum(p, axis=0, keepdims=True)
    pq = jnp.sum(p * p, axis=0, keepdims=True)

    @pl.when(i == 0)
    def _():
        accS[...] = ps
        accQ[...] = pq

    @pl.when(i > 0)
    def _():
        accS[...] += ps
        accQ[...] += pq

    @pl.when(i == NB - 1)
    def _():
        s_ref[...] = accS[...]
        q_ref[...] = accQ[...]


def _k1(x, W_in, b_in):
    return pl.pallas_call(
        _k1_body,
        grid=(NB,),
        in_specs=[
            pl.BlockSpec((R, H), lambda i: (i, 0)),
            pl.BlockSpec((H, H), lambda i: (0, 0)),
            pl.BlockSpec((1, H), lambda i: (0, 0)),
        ],
        out_specs=[
            pl.BlockSpec((R, H), lambda i: (i, 0)),
            pl.BlockSpec((1, H), lambda i: (0, 0)),
            pl.BlockSpec((1, H), lambda i: (0, 0)),
        ],
        out_shape=[
            jax.ShapeDtypeStruct((N, H), jnp.float32),
            jax.ShapeDtypeStruct((1, H), jnp.float32),
            jax.ShapeDtypeStruct((1, H), jnp.float32),
        ],
        scratch_shapes=[
            pltpu.VMEM((1, H), jnp.float32),
            pltpu.VMEM((1, H), jnp.float32),
        ],
    )(x, W_in, b_in)


def _k3_body(p_ref, s_ref, q_ref, g_ref, be_ref, degA, degB, w_ref, batch_ref,
             h0_ref, y_ref, dinv_ref, xs_ref, cnt_ref, accXS, accCNT):
    i = pl.program_id(0)
    c = pl.program_id(1)
    h0 = _bn_apply(p_ref[...], s_ref, q_ref, g_ref, be_ref)
    d = degA[0, :, 0:1] + degB[0, :, 0:1] + 1.0
    dinv = lax.rsqrt(d)
    y = jnp.dot(h0, w_ref[...], preferred_element_type=jnp.float32) * dinv
    y_ref[0] = y

    @pl.when(c == 0)
    def _():
        h0_ref[...] = h0
        dinv_ref[...] = dinv
        onehot = (batch_ref[...] == lax.broadcasted_iota(jnp.int32, (1, NG), 1)
                  ).astype(jnp.float32)
        pxs = lax.dot_general(onehot, h0, (((0,), (0,)), ((), ())),
                              preferred_element_type=jnp.float32)
        pcnt = lax.dot_general(onehot, jnp.ones((R, 1), jnp.float32),
                               (((0,), (0,)), ((), ())),
                               preferred_element_type=jnp.float32)

        @pl.when(i == 0)
        def _():
            accXS[...] = pxs
            accCNT[...] = pcnt

        @pl.when(i > 0)
        def _():
            accXS[...] += pxs
            accCNT[...] += pcnt

    @pl.when((i == NB - 1) & (c == 1))
    def _():
        xs_ref[...] = accXS[...]
        cnt_ref[...] = accCNT[...]


def _k3(P, S, Q, g, be, degp, W1, batch2d):
    return pl.pallas_call(
        _k3_body,
        grid=(NB, 2),
        in_specs=[
            pl.BlockSpec((R, H), lambda i, c: (i, 0)),
            pl.BlockSpec((1, H), lambda i, c: (0, 0)),
            pl.BlockSpec((1, H), lambda i, c: (0, 0)),
            pl.BlockSpec((1, H), lambda i, c: (0, 0)),
            pl.BlockSpec((1, H), lambda i, c: (0, 0)),
            pl.BlockSpec((1, R, HH), lambda i, c: (0, i, 0)),
            pl.BlockSpec((1, R, HH), lambda i, c: (1, i, 0)),
            pl.BlockSpec((H, HH), lambda i, c: (0, c)),
            pl.BlockSpec((R, 1), lambda i, c: (i, 0)),
        ],
        out_specs=[
            pl.BlockSpec((R, H), lambda i, c: (i, 0)),
            pl.BlockSpec((1, R, HH), lambda i, c: (c, i, 0)),
            pl.BlockSpec((R, 1), lambda i, c: (i, 0)),
            pl.BlockSpec((NG, H), lambda i, c: (0, 0)),
            pl.BlockSpec((NG, 1), lambda i, c: (0, 0)),
        ],
        out_shape=[
            jax.ShapeDtypeStruct((N, H), jnp.float32),
            jax.ShapeDtypeStruct((2, NP, HH), jnp.float32),
            jax.ShapeDtypeStruct((N, 1), jnp.float32),
            jax.ShapeDtypeStruct((NG, H), jnp.float32),
            jax.ShapeDtypeStruct((NG, 1), jnp.float32),
        ],
        scratch_shapes=[
            pltpu.VMEM((NG, H), jnp.float32),
            pltpu.VMEM((NG, 1), jnp.float32),
        ],
    )(P, S, Q, g, be, degp, degp, W1, batch2d)


def _k5_body(aggA, aggB, dinv_ref, b_ref, s_ref, q_ref, accS, accQ):
    i = pl.program_id(0)
    agg = jnp.concatenate([aggA[0], aggB[0]], axis=1)
    cp = agg * dinv_ref[...] + b_ref[...]
    ps = jnp.sum(cp, axis=0, keepdims=True)
    pq = jnp.sum(cp * cp, axis=0, keepdims=True)

    @pl.when(i == 0)
    def _():
        accS[...] = ps
        accQ[...] = pq

    @pl.when(i > 0)
    def _():
        accS[...] += ps
        accQ[...] += pq

    @pl.when(i == NB - 1)
    def _():
        s_ref[...] = accS[...]
        q_ref[...] = accQ[...]


def _k5(agg3, dinv, b):
    return pl.pallas_call(
        _k5_body,
        grid=(NB,),
        in_specs=[
            pl.BlockSpec((1, R, HH), lambda i: (0, i, 0)),
            pl.BlockSpec((1, R, HH), lambda i: (1, i, 0)),
            pl.BlockSpec((R, 1), lambda i: (i, 0)),
            pl.BlockSpec((1, H), lambda i: (0, 0)),
        ],
        out_specs=[
            pl.BlockSpec((1, H), lambda i: (0, 0)),
            pl.BlockSpec((1, H), lambda i: (0, 0)),
        ],
        out_shape=[
            jax.ShapeDtypeStruct((1, H), jnp.float32),
            jax.ShapeDtypeStruct((1, H), jnp.float32),
        ],
        scratch_shapes=[
            pltpu.VMEM((1, H), jnp.float32),
            pltpu.VMEM((1, H), jnp.float32),
        ],
    )(agg3, agg3, dinv, b)


def _k6_body(aggA, aggB, b_ref, s_ref, q_ref, g_ref, be_ref, hprev_ref,
             w_ref, dinv_ref, h1_ref, y_ref):
    c = pl.program_id(1)
    cp = jnp.concatenate([aggA[0], aggB[0]], axis=1) * dinv_ref[...] + b_ref[...]
    hb = _gelu(_bn_apply(cp, s_ref, q_ref, g_ref, be_ref))
    h1 = hb + hprev_ref[...]
    y = jnp.dot(h1, w_ref[...], preferred_element_type=jnp.float32) * dinv_ref[...]
    y_ref[0] = y

    @pl.when(c == 0)
    def _():
        h1_ref[...] = h1


def _k6(agg3, b, S, Q, g, be, hprev, W, dinv):
    return pl.pallas_call(
        _k6_body,
        grid=(NB, 2),
        in_specs=[
            pl.BlockSpec((1, R, HH), lambda i, c: (0, i, 0)),
            pl.BlockSpec((1, R, HH), lambda i, c: (1, i, 0)),
            pl.BlockSpec((1, H), lambda i, c: (0, 0)),
            pl.BlockSpec((1, H), lambda i, c: (0, 0)),
            pl.BlockSpec((1, H), lambda i, c: (0, 0)),
            pl.BlockSpec((1, H), lambda i, c: (0, 0)),
            pl.BlockSpec((1, H), lambda i, c: (0, 0)),
            pl.BlockSpec((R, H), lambda i, c: (i, 0)),
            pl.BlockSpec((H, HH), lambda i, c: (0, c)),
            pl.BlockSpec((R, 1), lambda i, c: (i, 0)),
        ],
        out_specs=[
            pl.BlockSpec((R, H), lambda i, c: (i, 0)),
            pl.BlockSpec((1, R, HH), lambda i, c: (c, i, 0)),
        ],
        out_shape=[
            jax.ShapeDtypeStruct((N, H), jnp.float32),
            jax.ShapeDtypeStruct((2, NP, HH), jnp.float32),
        ],
    )(agg3, agg3, b, S, Q, g, be, hprev, W, dinv)


def _k8b_body(aggA, aggB, b_ref, dinv_ref, s_ref, q_ref, g_ref, be_ref,
              hprev_ref, batch_ref, xg_ref, accXG):
    i = pl.program_id(0)
    cp = jnp.concatenate([aggA[0], aggB[0]], axis=1) * dinv_ref[...] + b_ref[...]
    hb = _gelu(_bn_apply(cp, s_ref, q_ref, g_ref, be_ref))
    h2 = hb + hprev_ref[...]
    onehot = (batch_ref[...] == lax.broadcasted_iota(jnp.int32, (1, NG), 1)
              ).astype(jnp.float32)
    pxg = lax.dot_general(onehot, h2, (((0,), (0,)), ((), ())),
                          preferred_element_type=jnp.float32)

    @pl.when(i == 0)
    def _():
        accXG[...] = pxg

    @pl.when(i > 0)
    def _():
        accXG[...] += pxg

    @pl.when(i == NB - 1)
    def _():
        xg_ref[...] = accXG[...]


def _k8b(agg3, b, dinv, S, Q, g, be, hprev, batch2d):
    return pl.pallas_call(
        _k8b_body,
        grid=(NB,),
        in_specs=[
            pl.BlockSpec((1, R, HH), lambda i: (0, i, 0)),
            pl.BlockSpec((1, R, HH), lambda i: (1, i, 0)),
            pl.BlockSpec((1, H), lambda i: (0, 0)),
            pl.BlockSpec((R, 1), lambda i: (i, 0)),
            pl.BlockSpec((1, H), lambda i: (0, 0)),
            pl.BlockSpec((1, H), lambda i: (0, 0)),
            pl.BlockSpec((1, H), lambda i: (0, 0)),
            pl.BlockSpec((1, H), lambda i: (0, 0)),
            pl.BlockSpec((R, H), lambda i: (i, 0)),
            pl.BlockSpec((R, 1), lambda i: (i, 0)),
        ],
        out_specs=[pl.BlockSpec((NG, H), lambda i: (0, 0))],
        out_shape=[jax.ShapeDtypeStruct((NG, H), jnp.float32)],
        scratch_shapes=[pltpu.VMEM((NG, H), jnp.float32)],
    )(agg3, agg3, b, dinv, S, Q, g, be, hprev, batch2d)


def _k9_body(xg_ref, xs_ref, cnt_ref, wf_ref, bf_ref, wc1_ref, bc1_ref,
             wc2_ref, bc2_ref, o_ref):
    inv = 1.0 / jnp.maximum(cnt_ref[...], 1.0)
    xg = xg_ref[...] * inv
    xs = xs_ref[...] * inv
    f = jnp.dot(xg, wf_ref[0:H], preferred_element_type=jnp.float32) + \
        jnp.dot(xs, wf_ref[H:2 * H], preferred_element_type=jnp.float32) + \
        bf_ref[...]
    f = _gelu(f)
    l1 = _gelu(jnp.dot(f, wc1_ref[...], preferred_element_type=jnp.float32)
               + bc1_ref[...])
    logits = jnp.dot(l1, wc2_ref[...], preferred_element_type=jnp.float32) \
        + bc2_ref[...]
    mx = jnp.max(logits, axis=1, keepdims=True)
    lse = jnp.log(jnp.sum(jnp.exp(logits - mx), axis=1, keepdims=True)) + mx
    o_ref[...] = logits - lse


def _k9(xg, xs, cnt, Wf, bf, Wc1, bc1, Wc2, bc2):
    return pl.pallas_call(
        _k9_body,
        out_shape=jax.ShapeDtypeStruct((NG, 20), jnp.float32),
    )(xg, xs, cnt, Wf, bf, Wc1, bc1, Wc2, bc2)


def kernel(x, edge_index, batch, W_in, b_in, g_in, be_in, W1, b1, g1, be1,
           W2, b2, g2, be2, Wf, bf, Wc1, bc1, Wc2, bc2):
    f32 = jnp.float32
    src = edge_index[0]
    dst = edge_index[1]
    batch2d = batch.reshape(N, 1)
    src3 = _pad_edges(src)
    dst3 = _pad_edges(dst)
    srco = jnp.concatenate([src3[None], src3[None] + NP],
                           axis=0).reshape(2 * NSUB * ECH_NCH, ECH)
    ones_rows = jnp.ones((ECH, HH), f32)
    zero_rows = jnp.zeros((RPT, HH), f32)

    P, S0, Q0 = _k1(x, W_in, b_in.reshape(1, H))

    degp = _sc_deg(dst3, ones_rows, zero_rows).reshape(2, NP, HH)

    h0, y1tab, dinv, xs_sum, cnt = _k3(
        P, S0, Q0, g_in.reshape(1, H), be_in.reshape(1, H), degp, W1, batch2d)

    agg1 = _sc_agg(y1tab.reshape(2 * NP, HH), srco, dst3).reshape(2, NP, HH)

    S1, Q1 = _k5(agg1, dinv, b1.reshape(1, H))

    h1, y2tab = _k6(agg1, b1.reshape(1, H), S1, Q1, g1.reshape(1, H),
                    be1.reshape(1, H), h0, W2, dinv)

    agg2 = _sc_agg(y2tab.reshape(2 * NP, HH), srco, dst3).reshape(2, NP, HH)

    S2, Q2 = _k5(agg2, dinv, b2.reshape(1, H))

    xg_sum = _k8b(agg2, b2.reshape(1, H), dinv, S2, Q2, g2.reshape(1, H),
                  be2.reshape(1, H), h1, batch2d)[0]

    return _k9(xg_sum, xs_sum, cnt, Wf, bf.reshape(1, H), Wc1,
               bc1.reshape(1, H // 2), Wc2, bc2.reshape(1, 20))
